# baseline (speedup 1.0000x reference)
import functools

import numpy as np
import jax
import jax.numpy as jnp
from jax import lax
from jax.experimental import pallas as pl
from jax.experimental.pallas import tpu as pltpu
from jax.experimental.pallas import tpu_sc as plsc
from jax.experimental import layout as jex_layout

N_WORDS_ = 8192
TOP_K_ = 50
EMB_ = 64
KPAD_ = 128
TOTAL_ = N_WORDS_ * TOP_K_
TOTP_ = N_WORDS_ * KPAD_

NC_ = 2
NS_ = 16
NW_ = NC_ * NS_
LANES_ = 16

ROWS_W_ = N_WORDS_ // NW_
CROWS_ = 16
NCHUNK_ = ROWS_W_ // CROWS_
ILEN_ = CROWS_ * KPAD_
RVLEN_ = CROWS_ * TOP_K_ * EMB_

_GROUPS = ((0, 16), (16, 16), (32, 16), (48, 2))


def _body(ti_hbm, emb_hbm, out_hbm,
          idx_a, idx_b, bkt_a, bkt_b, rows_a, rows_b, table_v,
          isem_a, isem_b, osem_a, osem_b):
  wid = lax.axis_index("s") * NC_ + lax.axis_index("c")
  row0 = wid * ROWS_W_
  lbase = wid * ROWS_W_ * TOP_K_

  pltpu.sync_copy(emb_hbm, table_v)

  def start_idx(c, idx_v, isem):
    crow = row0 + (c % NCHUNK_) * CROWS_
    pltpu.async_copy(ti_hbm.at[pl.ds(crow * KPAD_, ILEN_)], idx_v, isem)

  def compute(c, idx_v, bkt_v):
    crow = row0 + c * CROWS_

    def row_body(r, carry):
      w = crow + r
      for c0, _ in _GROUPS:
        t = idx_v[pl.ds(r * KPAD_ + c0, LANES_)]
        d = jnp.maximum(w - t, 1)
        b = jnp.where(d >= 2, 1, 0)
        for thr in (3, 4, 5, 8, 16, 32, 64):
          b = b + jnp.where(d >= thr, 1, 0)
        bkt_v[pl.ds(r * (4 * LANES_) + c0, LANES_)] = b
      return carry

    lax.fori_loop(0, CROWS_, row_body, 0)

  def process(j, c, idx_v, bkt_v, rows_v, isem, osem):
    crow = row0 + c * CROWS_
    pltpu.make_async_copy(ti_hbm.at[pl.ds(crow * KPAD_, ILEN_)],
                          idx_v, isem).wait()
    compute(c, idx_v, bkt_v)
    start_idx(c + 2, idx_v, isem)

    @pl.when(j > 0)
    def _():
      pltpu.make_async_copy(rows_v, out_hbm.at[pl.ds(crow, CROWS_)],
                            osem).wait()

    @plsc.parallel_loop(0, CROWS_, unroll=1)
    def _(r):
      for c0, nl in _GROUPS:
        bv = bkt_v[pl.ds(r * (4 * LANES_) + c0, LANES_)] * EMB_
        for l in range(nl):
          s = bv[l]
          for q in range(0, EMB_, LANES_):
            rows_v[r, c0 + l, pl.ds(q, LANES_)] = table_v[pl.ds(s + q, LANES_)]

    pltpu.async_copy(rows_v, out_hbm.at[pl.ds(crow, CROWS_)], osem)

  start_idx(0, idx_a, isem_a)
  start_idx(1, idx_b, isem_b)

  def chunk_pair(j, carry):
    process(j, 2 * j, idx_a, bkt_a, rows_a, isem_a, osem_a)
    process(j, 2 * j + 1, idx_b, bkt_b, rows_b, isem_b, osem_b)
    return carry

  lax.fori_loop(0, NCHUNK_ // 2, chunk_pair, 0)

  pltpu.make_async_copy(rows_a, out_hbm.at[pl.ds(row0, CROWS_)],
                        osem_a).wait()
  pltpu.make_async_copy(rows_b, out_hbm.at[pl.ds(row0, CROWS_)],
                        osem_b).wait()
  pltpu.make_async_copy(ti_hbm.at[pl.ds(row0 * KPAD_, ILEN_)],
                        idx_a, isem_a).wait()
  pltpu.make_async_copy(ti_hbm.at[pl.ds(row0 * KPAD_, ILEN_)],
                        idx_b, isem_b).wait()


@jax.jit
def kernel(top_indices, distance_emb):
  ti_pad = jnp.pad(top_indices.astype(jnp.int32),
                   ((0, 0), (0, KPAD_ - TOP_K_)))
  ti_flat = ti_pad.reshape(TOTP_)
  emb_flat = distance_emb.reshape(9 * EMB_)
  run = pl.kernel(
      _body,
      out_type=jax.ShapeDtypeStruct((N_WORDS_, TOP_K_, EMB_), jnp.float32),
      mesh=plsc.VectorSubcoreMesh(core_axis_name="c", subcore_axis_name="s"),
      scratch_types=[
          pltpu.VMEM((ILEN_,), jnp.int32),
          pltpu.VMEM((ILEN_,), jnp.int32),
          pltpu.VMEM((CROWS_ * 4 * LANES_,), jnp.int32),
          pltpu.VMEM((CROWS_ * 4 * LANES_,), jnp.int32),
          pltpu.VMEM((CROWS_, TOP_K_, EMB_), jnp.float32),
          pltpu.VMEM((CROWS_, TOP_K_, EMB_), jnp.float32),
          pltpu.VMEM((9 * EMB_,), jnp.float32),
          pltpu.SemaphoreType.DMA,
          pltpu.SemaphoreType.DMA,
          pltpu.SemaphoreType.DMA,
          pltpu.SemaphoreType.DMA,
      ],
      compiler_params=pltpu.CompilerParams(use_tc_tiling_on_sc=False),
  )
  return run(ti_flat, emb_flat)


# speedup vs baseline: 1.8912x; 1.8912x over previous
import functools

import numpy as np
import jax
import jax.numpy as jnp
from jax import lax
from jax.experimental import pallas as pl
from jax.experimental.pallas import tpu as pltpu
from jax.experimental.pallas import tpu_sc as plsc

N_WORDS_ = 8192
TOP_K_ = 50
EMB_ = 64
KPAD_ = 64

NC_ = 2
NS_ = 16
NW_ = NC_ * NS_
LANES_ = 16

ROWS_W_ = N_WORDS_ // NW_
CROWS_ = 16
NCHUNK_ = ROWS_W_ // CROWS_

_THRESHOLDS = (2, 3, 4, 5, 8, 16, 32, 64)

_GDN = lax.GatherDimensionNumbers(
    offset_dims=(), collapsed_slice_dims=(0,), start_index_map=(0,))


def _vgather(tv, b):
  return lax.gather(tv, b[:, None], dimension_numbers=_GDN,
                    slice_sizes=(1,),
                    mode=lax.GatherScatterMode.PROMISE_IN_BOUNDS)


def _body(ti_hbm, embt_hbm, out_hbm,
          idx_a, idx_b, bkt_v, rows_a, rows_b, table_v,
          isem_a, isem_b, osem_a, osem_b):
  wid = lax.axis_index("s") * NC_ + lax.axis_index("c")
  i0w = wid * ROWS_W_
  iota = lax.iota(jnp.int32, LANES_)

  pltpu.sync_copy(embt_hbm, table_v)

  def start_idx(c, idx_v, isem):
    ci = i0w + (c % NCHUNK_) * CROWS_
    pltpu.async_copy(ti_hbm.at[:, pl.ds(ci, CROWS_)], idx_v, isem)

  def process(j, c, idx_v, rows_v, isem, osem):
    ci = i0w + c * CROWS_
    wv = ci + iota
    pltpu.make_async_copy(ti_hbm.at[:, pl.ds(ci, CROWS_)],
                          idx_v, isem).wait()

    def bkt_body(k, carry):
      t = idx_v[k, :]
      d = jnp.maximum(wv - t, 1)
      b = jnp.where(d >= 2, 1, 0)
      for thr in _THRESHOLDS[1:]:
        b = b + jnp.where(d >= thr, 1, 0)
      bkt_v[k, :] = b
      return carry

    lax.fori_loop(0, TOP_K_, bkt_body, 0)
    start_idx(c + 2, idx_v, isem)

    @pl.when(j > 0)
    def _():
      pltpu.make_async_copy(
          rows_v, out_hbm.at[:, :, pl.ds(ci, CROWS_)], osem).wait()

    @plsc.parallel_loop(0, EMB_, unroll=2)
    def _(e):
      tv = table_v[e, pl.ds(0, LANES_)]
      for k in range(TOP_K_):
        rows_v[k, e, :] = _vgather(tv, bkt_v[k, :])

    pltpu.async_copy(rows_v, out_hbm.at[:, :, pl.ds(ci, CROWS_)], osem)

  start_idx(0, idx_a, isem_a)
  start_idx(1, idx_b, isem_b)

  def chunk_pair(j, carry):
    process(j, 2 * j, idx_a, rows_a, isem_a, osem_a)
    process(j, 2 * j + 1, idx_b, rows_b, isem_b, osem_b)
    return carry

  lax.fori_loop(0, NCHUNK_ // 2, chunk_pair, 0)

  pltpu.make_async_copy(rows_a, out_hbm.at[:, :, pl.ds(i0w, CROWS_)],
                        osem_a).wait()
  pltpu.make_async_copy(rows_b, out_hbm.at[:, :, pl.ds(i0w, CROWS_)],
                        osem_b).wait()
  pltpu.make_async_copy(ti_hbm.at[:, pl.ds(i0w, CROWS_)],
                        idx_a, isem_a).wait()
  pltpu.make_async_copy(ti_hbm.at[:, pl.ds(i0w, CROWS_)],
                        idx_b, isem_b).wait()


@jax.jit
def kernel(top_indices, distance_emb):
  ti_t = jnp.pad(top_indices.astype(jnp.int32),
                 ((0, 0), (0, KPAD_ - TOP_K_))).T
  emb_t = jnp.pad(distance_emb.T, ((0, 0), (0, 128 - 9)))
  run = pl.kernel(
      _body,
      out_type=jax.ShapeDtypeStruct((TOP_K_, EMB_, N_WORDS_), jnp.float32),
      mesh=plsc.VectorSubcoreMesh(core_axis_name="c", subcore_axis_name="s"),
      scratch_types=[
          pltpu.VMEM((KPAD_, CROWS_), jnp.int32),
          pltpu.VMEM((KPAD_, CROWS_), jnp.int32),
          pltpu.VMEM((TOP_K_, CROWS_), jnp.int32),
          pltpu.VMEM((TOP_K_, EMB_, CROWS_), jnp.float32),
          pltpu.VMEM((TOP_K_, EMB_, CROWS_), jnp.float32),
          pltpu.VMEM((EMB_, 128), jnp.float32),
          pltpu.SemaphoreType.DMA,
          pltpu.SemaphoreType.DMA,
          pltpu.SemaphoreType.DMA,
          pltpu.SemaphoreType.DMA,
      ],
      compiler_params=pltpu.CompilerParams(use_tc_tiling_on_sc=False),
  )
  out = run(ti_t, emb_t)
  return out.transpose(2, 0, 1)


# speedup vs baseline: 3.1506x; 1.6659x over previous
import functools

import numpy as np
import jax
import jax.numpy as jnp
from jax import lax
from jax.experimental import pallas as pl
from jax.experimental.pallas import tpu as pltpu
from jax.experimental.pallas import tpu_sc as plsc

N_WORDS_ = 8192
TOP_K_ = 50
EMB_ = 64
KPAD_ = 64

NC_ = 2
NS_ = 16
NW_ = NC_ * NS_
LANES_ = 16

ROWS_W_ = N_WORDS_ // NW_
CROWS_ = 16
NCHUNK_ = ROWS_W_ // CROWS_

_THRESHOLDS = (2, 3, 4, 5, 8, 16, 32, 64)

_GDN = lax.GatherDimensionNumbers(
    offset_dims=(), collapsed_slice_dims=(0,), start_index_map=(0,))


def _vgather(tv, b):
  return lax.gather(tv, b[:, None], dimension_numbers=_GDN,
                    slice_sizes=(1,),
                    mode=lax.GatherScatterMode.PROMISE_IN_BOUNDS)


def _body(ti_hbm, embt_hbm, out_hbm,
          idx_a, idx_b, bkt_v, rows_a, rows_b, table_v,
          isem_a, isem_b, osem_a, osem_b):
  wid = lax.axis_index("s") * NC_ + lax.axis_index("c")
  i0w = wid * ROWS_W_
  iota = lax.iota(jnp.int32, LANES_)

  pltpu.sync_copy(embt_hbm, table_v)

  def start_idx(c, idx_v, isem):
    ci = i0w + (c % NCHUNK_) * CROWS_
    pltpu.async_copy(ti_hbm.at[:, pl.ds(ci, CROWS_)], idx_v, isem)

  def process(j, c, idx_v, rows_v, isem, osem):
    ci = i0w + c * CROWS_
    it = lax.shift_right_logical(ci, 7)
    ii = pl.multiple_of(lax.bitwise_and(ci, 127), CROWS_)
    wv = ci + iota
    pltpu.make_async_copy(ti_hbm.at[:, pl.ds(ci, CROWS_)],
                          idx_v, isem).wait()

    def bkt_body(k, carry):
      t = idx_v[k, :]
      d = jnp.maximum(wv - t, 1)
      b = jnp.where(d >= 2, 1, 0)
      for thr in _THRESHOLDS[1:]:
        b = b + jnp.where(d >= thr, 1, 0)
      bkt_v[k, :] = b
      return carry

    lax.fori_loop(0, TOP_K_, bkt_body, 0)
    start_idx(c + 2, idx_v, isem)

    @pl.when(j > 0)
    def _():
      pltpu.make_async_copy(
          rows_v, out_hbm.at[:, :, it, :, pl.ds(ii, CROWS_)], osem).wait()

    @plsc.parallel_loop(0, EMB_, unroll=2)
    def _(e):
      tv = table_v[e, pl.ds(0, LANES_)]
      et = lax.shift_right_logical(e, 3)
      ei = lax.bitwise_and(e, 7)
      for k in range(TOP_K_):
        rows_v[k, et, ei, :] = _vgather(tv, bkt_v[k, :])

    pltpu.async_copy(rows_v, out_hbm.at[:, :, it, :, pl.ds(ii, CROWS_)],
                     osem)

  start_idx(0, idx_a, isem_a)
  start_idx(1, idx_b, isem_b)

  def chunk_pair(j, carry):
    process(j, 2 * j, idx_a, rows_a, isem_a, osem_a)
    process(j, 2 * j + 1, idx_b, rows_b, isem_b, osem_b)
    return carry

  lax.fori_loop(0, NCHUNK_ // 2, chunk_pair, 0)

  it0 = lax.shift_right_logical(i0w, 7)
  ii0 = pl.multiple_of(lax.bitwise_and(i0w, 127), CROWS_)
  pltpu.make_async_copy(rows_a,
                        out_hbm.at[:, :, it0, :, pl.ds(ii0, CROWS_)],
                        osem_a).wait()
  pltpu.make_async_copy(rows_b,
                        out_hbm.at[:, :, it0, :, pl.ds(ii0, CROWS_)],
                        osem_b).wait()
  pltpu.make_async_copy(ti_hbm.at[:, pl.ds(i0w, CROWS_)],
                        idx_a, isem_a).wait()
  pltpu.make_async_copy(ti_hbm.at[:, pl.ds(i0w, CROWS_)],
                        idx_b, isem_b).wait()


@jax.jit
def kernel(top_indices, distance_emb):
  ti_t = jnp.pad(top_indices.astype(jnp.int32),
                 ((0, 0), (0, KPAD_ - TOP_K_))).T
  emb_t = jnp.pad(distance_emb.T, ((0, 0), (0, 128 - 9)))
  run = pl.kernel(
      _body,
      out_type=jax.ShapeDtypeStruct((TOP_K_, EMB_ // 8, N_WORDS_ // 128,
                                     8, 128), jnp.float32),
      mesh=plsc.VectorSubcoreMesh(core_axis_name="c", subcore_axis_name="s"),
      scratch_types=[
          pltpu.VMEM((KPAD_, CROWS_), jnp.int32),
          pltpu.VMEM((KPAD_, CROWS_), jnp.int32),
          pltpu.VMEM((TOP_K_, CROWS_), jnp.int32),
          pltpu.VMEM((TOP_K_, EMB_ // 8, 8, CROWS_), jnp.float32),
          pltpu.VMEM((TOP_K_, EMB_ // 8, 8, CROWS_), jnp.float32),
          pltpu.VMEM((EMB_, 128), jnp.float32),
          pltpu.SemaphoreType.DMA,
          pltpu.SemaphoreType.DMA,
          pltpu.SemaphoreType.DMA,
          pltpu.SemaphoreType.DMA,
      ],
      compiler_params=pltpu.CompilerParams(use_tc_tiling_on_sc=False),
  )
  out = run(ti_t, emb_t)
  return out.transpose(2, 4, 0, 1, 3).reshape(N_WORDS_, TOP_K_, EMB_)
